# Initial kernel scaffold; baseline (speedup 1.0000x reference)
import jax
import jax.numpy as jnp
from jax.experimental import pallas as pl


def kernel(x, Wr, br, W1, b1, W2, b2):
    raise NotImplementedError("write your pallas kernel here")


# speedup vs baseline: 5.4587x; 5.4587x over previous
import functools

import jax
import jax.numpy as jnp
from jax import lax
from jax.experimental import pallas as pl
from jax.experimental.pallas import tpu as pltpu
from jax.experimental.pallas import tpu_sc as plsc

E, H, F, S = 8, 768, 3072, 512
T = 128
NB = 11
NP = NB * T

_NC = 2
_NS = 16
_NW = _NC * _NS
_TPW = S // _NW


def _router_plan_body(x_ref, wr_ref, br_ref, pos_ref, eb_ref, scale_ref):
    x = x_ref[...]
    s = jnp.dot(x, wr_ref[...],
                preferred_element_type=jnp.float32) + br_ref[...]
    m = jnp.max(s, axis=1, keepdims=True)
    z = jnp.sum(jnp.exp(s - m), axis=1, keepdims=True)
    ccoef = jnp.sum(z)
    iota_e = lax.broadcasted_iota(jnp.int32, (S, E), 1)
    eid = jnp.min(jnp.where(s >= m, iota_e, E), axis=1, keepdims=True)
    onehot = (iota_e == eid).astype(jnp.float32)
    r_i = lax.broadcasted_iota(jnp.int32, (S, S), 0)
    c_i = lax.broadcasted_iota(jnp.int32, (S, S), 1)
    ltri = (c_i <= r_i).astype(jnp.float32)
    csum = jnp.dot(ltri, onehot, preferred_element_type=jnp.float32)
    rank = jnp.sum(csum * onehot, axis=1, keepdims=True) - 1.0
    g = jnp.sum(onehot, axis=0, keepdims=True)
    nb_e = jnp.floor((g + (T - 1)) / T)
    ei = lax.broadcasted_iota(jnp.int32, (E, E), 0)
    ej = lax.broadcasted_iota(jnp.int32, (E, E), 1)
    stri = (ei < ej).astype(jnp.float32)
    bstart = jnp.dot(nb_e, stri, preferred_element_type=jnp.float32)
    rstart = bstart * T
    pos = jnp.sum(onehot * rstart, axis=1, keepdims=True) + rank
    pos_ref[...] = pos.astype(jnp.int32)
    bend = bstart + nb_e
    b_iota = lax.broadcasted_iota(jnp.int32, (1, NB), 1).astype(jnp.float32)
    eb = jnp.zeros((1, NB), jnp.int32)
    for e in range(E):
        eb = eb + (b_iota >= bend[:, e:e + 1]).astype(jnp.int32)
    eb_ref[...] = eb
    scale_ref[...] = jnp.broadcast_to((1.0 / z) * ccoef, (S, 128))


def _mlp_body(eb_ref, xb_ref, w1_ref, b1_ref, w2_ref, b2_ref, sc_ref, y_ref):
    del eb_ref
    h = jnp.dot(xb_ref[...], w1_ref[0], preferred_element_type=jnp.float32)
    h = h + b1_ref[0]
    h = 0.5 * h * (1.0 + lax.erf(h * 0.7071067811865476))
    y = jnp.dot(h, w2_ref[0], preferred_element_type=jnp.float32) + b2_ref[0]
    y_ref[...] = y * sc_ref[...][:, 0:1]


def _dispatch(x2, scale16, pos):
    mesh = plsc.VectorSubcoreMesh(core_axis_name="c", subcore_axis_name="s")

    @functools.partial(
        pl.kernel, mesh=mesh,
        out_type=[jax.ShapeDtypeStruct((NP, H), jnp.float32),
                  jax.ShapeDtypeStruct((NP, 128), jnp.float32)],
        scratch_types=[pltpu.VMEM((_TPW,), jnp.int32),
                       pltpu.VMEM((_TPW, H), jnp.float32),
                       pltpu.VMEM((_TPW, 128), jnp.float32),
                       pltpu.SemaphoreType.DMA,
                       pltpu.SemaphoreType.DMA],
    )
    def body(x_hbm, s16_hbm, pos_hbm, xpad_hbm, spad_hbm,
             idx_v, rows_v, srows_v, sem1, sem2):
        wid = lax.axis_index("s") * _NC + lax.axis_index("c")
        base = wid * _TPW
        pltpu.sync_copy(pos_hbm.at[pl.ds(base, _TPW)], idx_v)
        pltpu.sync_copy(x_hbm.at[pl.ds(base, _TPW)], rows_v)
        pltpu.sync_copy(s16_hbm.at[pl.ds(base, _TPW)], srows_v)
        cp1 = pltpu.async_copy(rows_v, xpad_hbm.at[idx_v], sem1)
        cp2 = pltpu.async_copy(srows_v, spad_hbm.at[idx_v], sem2)
        cp1.wait()
        cp2.wait()

    return body(x2, scale16, pos)


def _unsort(ypad, pos):
    mesh = plsc.VectorSubcoreMesh(core_axis_name="c", subcore_axis_name="s")

    @functools.partial(
        pl.kernel, mesh=mesh,
        out_type=jax.ShapeDtypeStruct((S, H), jnp.float32),
        scratch_types=[pltpu.VMEM((_TPW,), jnp.int32),
                       pltpu.VMEM((_TPW, H), jnp.float32),
                       pltpu.SemaphoreType.DMA],
    )
    def body(ypad_hbm, pos_hbm, out_hbm, idx_v, rows_v, sem):
        wid = lax.axis_index("s") * _NC + lax.axis_index("c")
        base = wid * _TPW
        pltpu.sync_copy(pos_hbm.at[pl.ds(base, _TPW)], idx_v)
        pltpu.async_copy(ypad_hbm.at[idx_v], rows_v, sem).wait()
        pltpu.sync_copy(rows_v, out_hbm.at[pl.ds(base, _TPW)])

    return body(ypad, pos)


def kernel(x, Wr, br, W1, b1, W2, b2):
    x2 = x.reshape(S, H)
    pos2, eb2, scale16 = pl.pallas_call(
        _router_plan_body,
        out_shape=[jax.ShapeDtypeStruct((S, 1), jnp.int32),
                   jax.ShapeDtypeStruct((1, NB), jnp.int32),
                   jax.ShapeDtypeStruct((S, 128), jnp.float32)],
    )(x2, Wr, br.reshape(1, E))
    pos = pos2.reshape(S)
    eb = eb2.reshape(NB)

    xpad, spad = _dispatch(x2, scale16, pos)

    grid_spec = pltpu.PrefetchScalarGridSpec(
        num_scalar_prefetch=1,
        grid=(NB,),
        in_specs=[
            pl.BlockSpec((T, H), lambda b, eb_r: (b, 0)),
            pl.BlockSpec((1, H, F), lambda b, eb_r: (eb_r[b], 0, 0)),
            pl.BlockSpec((1, 1, F), lambda b, eb_r: (eb_r[b], 0, 0)),
            pl.BlockSpec((1, F, H), lambda b, eb_r: (eb_r[b], 0, 0)),
            pl.BlockSpec((1, 1, H), lambda b, eb_r: (eb_r[b], 0, 0)),
            pl.BlockSpec((T, 128), lambda b, eb_r: (b, 0)),
        ],
        out_specs=pl.BlockSpec((T, H), lambda b, eb_r: (b, 0)),
    )
    ypad = pl.pallas_call(
        _mlp_body,
        grid_spec=grid_spec,
        out_shape=jax.ShapeDtypeStruct((NP, H), jnp.float32),
    )(eb, xpad, W1, b1.reshape(E, 1, F), W2, b2.reshape(E, 1, H), spad)

    out = _unsort(ypad, pos)
    return out.reshape(1, S, H)


# speedup vs baseline: 5.9150x; 1.0836x over previous
import functools

import jax
import jax.numpy as jnp
from jax import lax
from jax.experimental import pallas as pl
from jax.experimental.pallas import tpu as pltpu
from jax.experimental.pallas import tpu_sc as plsc

E, H, F, S = 8, 768, 3072, 512
T = 128
NB = 11
NP = NB * T

_NC = 2
_NS = 16
_NW = _NC * _NS
_TPW = S // _NW


def _router_plan_body(x_ref, wr_ref, br_ref, pos_ref, eb_ref, scale_ref):
    x = x_ref[...]
    s = jnp.dot(x, wr_ref[...],
                preferred_element_type=jnp.float32) + br_ref[...]
    m = jnp.max(s, axis=1, keepdims=True)
    z = jnp.sum(jnp.exp(s - m), axis=1, keepdims=True)
    ccoef = jnp.sum(z)
    iota_e = lax.broadcasted_iota(jnp.int32, (S, E), 1)
    eid = jnp.min(jnp.where(s >= m, iota_e, E), axis=1, keepdims=True)
    onehot = (iota_e == eid).astype(jnp.float32)
    r_i = lax.broadcasted_iota(jnp.int32, (S, S), 0)
    c_i = lax.broadcasted_iota(jnp.int32, (S, S), 1)
    ltri = (c_i <= r_i).astype(jnp.float32)
    csum = jnp.dot(ltri, onehot, preferred_element_type=jnp.float32)
    rank = jnp.sum(csum * onehot, axis=1, keepdims=True) - 1.0
    g = jnp.sum(onehot, axis=0, keepdims=True)
    nb_e = jnp.floor((g + (T - 1)) / T)
    ei = lax.broadcasted_iota(jnp.int32, (E, E), 0)
    ej = lax.broadcasted_iota(jnp.int32, (E, E), 1)
    stri = (ei < ej).astype(jnp.float32)
    bstart = jnp.dot(nb_e, stri, preferred_element_type=jnp.float32)
    rstart = bstart * T
    pos = jnp.sum(onehot * rstart, axis=1, keepdims=True) + rank
    pos_ref[...] = pos.astype(jnp.int32)
    bend = bstart + nb_e
    b_iota = lax.broadcasted_iota(jnp.int32, (1, NB), 1).astype(jnp.float32)
    eb = jnp.zeros((1, NB), jnp.int32)
    for e in range(E):
        eb = eb + (b_iota >= bend[:, e:e + 1]).astype(jnp.int32)
    eb = jnp.minimum(eb, E - 1)
    nact = jnp.sum(nb_e).astype(jnp.int32)
    eb_ref[...] = jnp.concatenate(
        [eb, jnp.broadcast_to(nact, (1, 1))], axis=1)
    scale_ref[...] = jnp.broadcast_to((1.0 / z) * ccoef, (S, 128))


def _mlp_body(eb_ref, xb_ref, w1_ref, b1_ref, w2_ref, b2_ref, sc_ref, y_ref):
    @pl.when(pl.program_id(0) < eb_ref[NB])
    def _():
        h = jnp.dot(xb_ref[...], w1_ref[0], preferred_element_type=jnp.float32)
        h = h + b1_ref[0]
        h = 0.5 * h * (1.0 + lax.erf(h * 0.7071067811865476))
        y = jnp.dot(h, w2_ref[0], preferred_element_type=jnp.float32) + b2_ref[0]
        y_ref[...] = y * sc_ref[...][:, 0:1]


def _dispatch(x2, scale16, pos):
    mesh = plsc.VectorSubcoreMesh(core_axis_name="c", subcore_axis_name="s")

    @functools.partial(
        pl.kernel, mesh=mesh,
        out_type=[jax.ShapeDtypeStruct((NP, H), jnp.float32),
                  jax.ShapeDtypeStruct((NP, 128), jnp.float32)],
        scratch_types=[pltpu.VMEM((_TPW,), jnp.int32),
                       pltpu.VMEM((_TPW, H), jnp.float32),
                       pltpu.VMEM((_TPW, 128), jnp.float32),
                       pltpu.SemaphoreType.DMA,
                       pltpu.SemaphoreType.DMA],
    )
    def body(x_hbm, s16_hbm, pos_hbm, xpad_hbm, spad_hbm,
             idx_v, rows_v, srows_v, sem1, sem2):
        wid = lax.axis_index("s") * _NC + lax.axis_index("c")
        base = wid * _TPW
        pltpu.sync_copy(pos_hbm.at[pl.ds(base, _TPW)], idx_v)
        pltpu.sync_copy(x_hbm.at[pl.ds(base, _TPW)], rows_v)
        pltpu.sync_copy(s16_hbm.at[pl.ds(base, _TPW)], srows_v)
        cp1 = pltpu.async_copy(rows_v, xpad_hbm.at[idx_v], sem1)
        cp2 = pltpu.async_copy(srows_v, spad_hbm.at[idx_v], sem2)
        cp1.wait()
        cp2.wait()

    return body(x2, scale16, pos)


def _unsort(ypad, pos):
    mesh = plsc.VectorSubcoreMesh(core_axis_name="c", subcore_axis_name="s")

    @functools.partial(
        pl.kernel, mesh=mesh,
        out_type=jax.ShapeDtypeStruct((S, H), jnp.float32),
        scratch_types=[pltpu.VMEM((_TPW,), jnp.int32),
                       pltpu.VMEM((_TPW, H), jnp.float32),
                       pltpu.SemaphoreType.DMA],
    )
    def body(ypad_hbm, pos_hbm, out_hbm, idx_v, rows_v, sem):
        wid = lax.axis_index("s") * _NC + lax.axis_index("c")
        base = wid * _TPW
        pltpu.sync_copy(pos_hbm.at[pl.ds(base, _TPW)], idx_v)
        pltpu.async_copy(ypad_hbm.at[idx_v], rows_v, sem).wait()
        pltpu.sync_copy(rows_v, out_hbm.at[pl.ds(base, _TPW)])

    return body(ypad, pos)


def kernel(x, Wr, br, W1, b1, W2, b2):
    x2 = x.reshape(S, H)
    pos2, eb2, scale16 = pl.pallas_call(
        _router_plan_body,
        out_shape=[jax.ShapeDtypeStruct((S, 1), jnp.int32),
                   jax.ShapeDtypeStruct((1, NB + 1), jnp.int32),
                   jax.ShapeDtypeStruct((S, 128), jnp.float32)],
    )(x2, Wr, br.reshape(1, E))
    pos = pos2.reshape(S)
    eb = eb2.reshape(NB + 1)

    xpad, spad = _dispatch(x2, scale16, pos)

    grid_spec = pltpu.PrefetchScalarGridSpec(
        num_scalar_prefetch=1,
        grid=(NB,),
        in_specs=[
            pl.BlockSpec((T, H), lambda b, eb_r: (b, 0)),
            pl.BlockSpec((1, H, F), lambda b, eb_r: (eb_r[b], 0, 0)),
            pl.BlockSpec((1, 1, F), lambda b, eb_r: (eb_r[b], 0, 0)),
            pl.BlockSpec((1, F, H), lambda b, eb_r: (eb_r[b], 0, 0)),
            pl.BlockSpec((1, 1, H), lambda b, eb_r: (eb_r[b], 0, 0)),
            pl.BlockSpec((T, 128), lambda b, eb_r: (b, 0)),
        ],
        out_specs=pl.BlockSpec((T, H), lambda b, eb_r: (b, 0)),
    )
    ypad = pl.pallas_call(
        _mlp_body,
        grid_spec=grid_spec,
        out_shape=jax.ShapeDtypeStruct((NP, H), jnp.float32),
    )(eb, xpad, W1, b1.reshape(E, 1, F), W2, b2.reshape(E, 1, H), spad)

    out = _unsort(ypad, pos)
    return out.reshape(1, S, H)
